# Initial kernel scaffold; baseline (speedup 1.0000x reference)
import jax
import jax.numpy as jnp
from jax.experimental import pallas as pl


def kernel(hidden_states, cos, sin, Wq, Wkv, Wo, q_norm_w, k_norm_w):
    raise NotImplementedError("write your pallas kernel here")


# speedup vs baseline: 43.4699x; 43.4699x over previous
import numpy as np
import jax
import jax.numpy as jnp
from jax.experimental import pallas as pl

_B, _T, _D = 1, 2048, 1024
_H, _HKV, _DH = 16, 8, 64
_SINK, _WIN, _TOPK = 16, 256, 256
_EPS = 1e-06
_SCALE = _DH ** -0.5
_NEG = float(np.finfo(np.float32).min)

_BT = 256
_BQ = 256
_NIT = 22
_HPP = 4


def _proj_body(hs_ref, wqT_ref, wqrT_ref, wkvT_ref, wkvrT_ref,
               cq_ref, sq_ref, ck_ref, sk_ref,
               wq_ref, wqr_ref, wk_ref, wkr_ref,
               eq_ref, exq_ref, ek_ref, exk_ref,
               q_ref, k_ref, v_ref):
    hs = hs_ref[...].astype(jnp.bfloat16)
    hp = jax.lax.Precision.HIGHEST
    qa = jnp.dot(hs, wqT_ref[...].astype(jnp.bfloat16),
                 preferred_element_type=jnp.float32)
    qb = jnp.dot(hs, wqrT_ref[...].astype(jnp.bfloat16),
                 preferred_element_type=jnp.float32)
    var_q = jnp.dot(qa * qa, eq_ref[...], preferred_element_type=jnp.float32,
                    precision=hp)
    rs_q = jnp.dot(jax.lax.rsqrt(var_q + _EPS), exq_ref[...],
                   preferred_element_type=jnp.float32, precision=hp)
    q_ref[...] = rs_q * (qa * wq_ref[...] * cq_ref[...] +
                         qb * wqr_ref[...] * sq_ref[...])
    ka = jnp.dot(hs, wkvT_ref[...].astype(jnp.bfloat16),
                 preferred_element_type=jnp.float32)
    kb = jnp.dot(hs, wkvrT_ref[...].astype(jnp.bfloat16),
                 preferred_element_type=jnp.float32)
    var_k = jnp.dot(ka * ka, ek_ref[...], preferred_element_type=jnp.float32,
                    precision=hp)
    rs_k = jnp.dot(jax.lax.rsqrt(var_k + _EPS), exk_ref[...],
                   preferred_element_type=jnp.float32, precision=hp)
    k_ref[...] = rs_k * (ka * wk_ref[...] * ck_ref[...] +
                         kb * wkr_ref[...] * sk_ref[...])
    v_ref[...] = ka


def _attn_body(q_ref, k_ref, v_ref, o_ref):
    row0 = pl.program_id(1) * _BQ
    rows = row0 + jax.lax.broadcasted_iota(jnp.int32, (_BQ, _T), 0)
    cols = jax.lax.broadcasted_iota(jnp.int32, (_BQ, _T), 1)
    base_keep = (cols < _SINK) | (jnp.abs(rows - cols) <= _WIN)
    causal = cols <= rows
    k2 = k_ref[...]
    v2 = v_ref[...]
    for a in range(_HPP):
        qh = q_ref[:, a * _DH:(a + 1) * _DH].astype(jnp.bfloat16)
        kv_off = (a // 2) * _DH
        kh = k2[:, kv_off:kv_off + _DH].astype(jnp.bfloat16)
        vh = v2[:, kv_off:kv_off + _DH].astype(jnp.bfloat16)
        s = jax.lax.dot_general(qh, kh, (((1,), (1,)), ((), ())),
                                preferred_element_type=jnp.float32) * _SCALE
        lo = jnp.min(s, axis=1, keepdims=True) - 1.0
        hi = jnp.max(s, axis=1, keepdims=True)

        def bs(_, c):
            lo_, hi_ = c
            mid = 0.5 * (lo_ + hi_)
            cnt = jnp.sum((s > mid).astype(jnp.float32), axis=1,
                          keepdims=True)
            pred = cnt >= _TOPK
            return jnp.where(pred, mid, lo_), jnp.where(pred, hi_, mid)

        lo, hi = jax.lax.fori_loop(0, _NIT, bs, (lo, hi))
        keep = (base_keep | (s > lo)) & causal
        m = jnp.max(jnp.where(keep, s, _NEG), axis=1, keepdims=True)
        p = jnp.where(keep, jnp.exp(s - m), 0.0)
        p = p / jnp.sum(p, axis=1, keepdims=True)
        o_ref[:, a * _DH:(a + 1) * _DH] = jax.lax.dot_general(
            p.astype(jnp.bfloat16), vh, (((1,), (0,)), ((), ())),
            preferred_element_type=jnp.float32)


def _oproj_body(x_ref, woT_ref, o_ref):
    o_ref[...] = jnp.dot(x_ref[...].astype(jnp.bfloat16),
                         woT_ref[...].astype(jnp.bfloat16),
                         preferred_element_type=jnp.float32)


def kernel(hidden_states, cos, sin, Wq, Wkv, Wo, q_norm_w, k_norm_w):
    f32 = jnp.float32
    hs = hidden_states.reshape(_T, _D)

    h2 = _DH // 2
    Wq3 = Wq.reshape(_H, _DH, _D)
    WqrT = jnp.concatenate([-Wq3[:, h2:], Wq3[:, :h2]],
                           axis=1).reshape(_H * _DH, _D).T
    Wkv3 = Wkv.reshape(_HKV, _DH, _D)
    WkvrT = jnp.concatenate([-Wkv3[:, h2:], Wkv3[:, :h2]],
                            axis=1).reshape(_HKV * _DH, _D).T
    WqT, WkvT, WoT = Wq.T, Wkv.T, Wo.T

    cq = jnp.tile(cos, (1, _H))
    sq = jnp.tile(sin, (1, _H))
    ck = jnp.tile(cos, (1, _HKV))
    sk = jnp.tile(sin, (1, _HKV))
    qw_rot = jnp.concatenate([q_norm_w[h2:], q_norm_w[:h2]])
    kw_rot = jnp.concatenate([k_norm_w[h2:], k_norm_w[:h2]])
    wq = jnp.tile(q_norm_w, _H)[None, :]
    wqr = jnp.tile(qw_rot, _H)[None, :]
    wk = jnp.tile(k_norm_w, _HKV)[None, :]
    wkr = jnp.tile(kw_rot, _HKV)[None, :]

    eq = jnp.asarray(np.kron(np.eye(_H), np.ones((_DH, 1))) / _DH, f32)
    exq = jnp.asarray(np.kron(np.eye(_H), np.ones((1, _DH))), f32)
    ek = jnp.asarray(np.kron(np.eye(_HKV), np.ones((_DH, 1))) / _DH, f32)
    exk = jnp.asarray(np.kron(np.eye(_HKV), np.ones((1, _DH))), f32)

    full = lambda shape: pl.BlockSpec(shape, lambda *_: tuple(0 for _ in shape))
    rowblk = lambda w: pl.BlockSpec((_BT, w), lambda i: (i, 0))

    q, k, v = pl.pallas_call(
        _proj_body,
        grid=(_T // _BT,),
        in_specs=[
            rowblk(_D),
            full((_D, _H * _DH)),
            full((_D, _H * _DH)),
            full((_D, _HKV * _DH)),
            full((_D, _HKV * _DH)),
            rowblk(_H * _DH),
            rowblk(_H * _DH),
            rowblk(_HKV * _DH),
            rowblk(_HKV * _DH),
            full((1, _H * _DH)),
            full((1, _H * _DH)),
            full((1, _HKV * _DH)),
            full((1, _HKV * _DH)),
            full((_H * _DH, _H)),
            full((_H, _H * _DH)),
            full((_HKV * _DH, _HKV)),
            full((_HKV, _HKV * _DH)),
        ],
        out_specs=[
            rowblk(_H * _DH),
            rowblk(_HKV * _DH),
            rowblk(_HKV * _DH),
        ],
        out_shape=[
            jax.ShapeDtypeStruct((_T, _H * _DH), f32),
            jax.ShapeDtypeStruct((_T, _HKV * _DH), f32),
            jax.ShapeDtypeStruct((_T, _HKV * _DH), f32),
        ],
    )(hs, WqT, WqrT, WkvT, WkvrT, cq, sq, ck, sk,
      wq, wqr, wk, wkr, eq, exq, ek, exk)

    attn = pl.pallas_call(
        _attn_body,
        grid=(_H // _HPP, _T // _BQ),
        in_specs=[
            pl.BlockSpec((_BQ, _HPP * _DH), lambda j, i: (i, j)),
            pl.BlockSpec((_T, 2 * _DH), lambda j, i: (0, j)),
            pl.BlockSpec((_T, 2 * _DH), lambda j, i: (0, j)),
        ],
        out_specs=pl.BlockSpec((_BQ, _HPP * _DH), lambda j, i: (i, j)),
        out_shape=jax.ShapeDtypeStruct((_T, _H * _DH), f32),
    )(q, k, v)

    out = pl.pallas_call(
        _oproj_body,
        grid=(_T // _BT,),
        in_specs=[rowblk(_H * _DH), full((_H * _DH, _D))],
        out_specs=rowblk(_D),
        out_shape=jax.ShapeDtypeStruct((_T, _D), f32),
    )(attn, WoT)

    return out.reshape(_B, _T, _D)


# speedup vs baseline: 55.5622x; 1.2782x over previous
import numpy as np
import jax
import jax.numpy as jnp
from jax.experimental import pallas as pl

_B, _T, _D = 1, 2048, 1024
_H, _HKV, _DH = 16, 8, 64
_SINK, _WIN, _TOPK = 16, 256, 256
_EPS = 1e-06
_SCALE = _DH ** -0.5
_NEG = float(np.finfo(np.float32).min)

_BT = 256
_BQ = 256
_NIT = 17
_HPP = 4


def _proj_body(hs_ref, wqT_ref, wqrT_ref, wkvT_ref, wkvrT_ref,
               cq_ref, sq_ref, ck_ref, sk_ref,
               wq_ref, wqr_ref, wk_ref, wkr_ref,
               eq_ref, exq_ref, ek_ref, exk_ref,
               q_ref, k_ref, v_ref):
    hs = hs_ref[...].astype(jnp.bfloat16)
    hp = jax.lax.Precision.HIGHEST
    qa = jnp.dot(hs, wqT_ref[...].astype(jnp.bfloat16),
                 preferred_element_type=jnp.float32)
    qb = jnp.dot(hs, wqrT_ref[...].astype(jnp.bfloat16),
                 preferred_element_type=jnp.float32)
    var_q = jnp.dot(qa * qa, eq_ref[...], preferred_element_type=jnp.float32,
                    precision=hp)
    rs_q = jnp.dot(jax.lax.rsqrt(var_q + _EPS), exq_ref[...],
                   preferred_element_type=jnp.float32, precision=hp)
    q_ref[...] = (rs_q * (qa * wq_ref[...] * cq_ref[...] +
                          qb * wqr_ref[...] * sq_ref[...])).astype(jnp.bfloat16)
    ka = jnp.dot(hs, wkvT_ref[...].astype(jnp.bfloat16),
                 preferred_element_type=jnp.float32)
    kb = jnp.dot(hs, wkvrT_ref[...].astype(jnp.bfloat16),
                 preferred_element_type=jnp.float32)
    var_k = jnp.dot(ka * ka, ek_ref[...], preferred_element_type=jnp.float32,
                    precision=hp)
    rs_k = jnp.dot(jax.lax.rsqrt(var_k + _EPS), exk_ref[...],
                   preferred_element_type=jnp.float32, precision=hp)
    k_ref[...] = (rs_k * (ka * wk_ref[...] * ck_ref[...] +
                          kb * wkr_ref[...] * sk_ref[...])).astype(jnp.bfloat16)
    v_ref[...] = ka.astype(jnp.bfloat16)


def _attn_body(q_ref, k_ref, v_ref, o_ref):
    row0 = pl.program_id(1) * _BQ
    rows = row0 + jax.lax.broadcasted_iota(jnp.int32, (_BQ, _T), 0)
    cols = jax.lax.broadcasted_iota(jnp.int32, (_BQ, _T), 1)
    base_keep = (cols < _SINK) | (jnp.abs(rows - cols) <= _WIN)
    causal = cols <= rows
    k2 = k_ref[...]
    v2 = v_ref[...]
    for a in range(_HPP):
        qh = q_ref[:, a * _DH:(a + 1) * _DH]
        kv_off = (a // 2) * _DH
        kh = k2[:, kv_off:kv_off + _DH]
        vh = v2[:, kv_off:kv_off + _DH]
        s = jax.lax.dot_general(qh, kh, (((1,), (1,)), ((), ())),
                                preferred_element_type=jnp.float32) * _SCALE
        def search():
            lo = jnp.min(s, axis=1, keepdims=True) - 1.0
            hi = jnp.max(s, axis=1, keepdims=True)

            def bs(_, c):
                lo_, hi_ = c
                mid = 0.5 * (lo_ + hi_)
                cnt = jnp.sum((s > mid).astype(jnp.float32), axis=1,
                              keepdims=True)
                pred = cnt >= _TOPK
                return jnp.where(pred, mid, lo_), jnp.where(pred, hi_, mid)

            return jax.lax.fori_loop(0, _NIT, bs, (lo, hi))[0]

        lo = jax.lax.cond(pl.program_id(1) == 0,
                          lambda: jnp.full((_BQ, 1), jnp.inf, jnp.float32),
                          search)
        keep = (base_keep | (s > lo)) & causal
        m = jnp.max(jnp.where(keep, s, _NEG), axis=1, keepdims=True)
        p = jnp.where(keep, jnp.exp(s - m), 0.0)
        p = p / jnp.sum(p, axis=1, keepdims=True)
        o_ref[:, a * _DH:(a + 1) * _DH] = jax.lax.dot_general(
            p.astype(jnp.bfloat16), vh, (((1,), (0,)), ((), ())),
            preferred_element_type=jnp.float32).astype(jnp.bfloat16)


def _oproj_body(x_ref, woT_ref, o_ref):
    o_ref[...] = jnp.dot(x_ref[...], woT_ref[...].astype(jnp.bfloat16),
                         preferred_element_type=jnp.float32)


def kernel(hidden_states, cos, sin, Wq, Wkv, Wo, q_norm_w, k_norm_w):
    f32 = jnp.float32
    hs = hidden_states.reshape(_T, _D)

    h2 = _DH // 2
    Wq3 = Wq.reshape(_H, _DH, _D)
    WqrT = jnp.concatenate([-Wq3[:, h2:], Wq3[:, :h2]],
                           axis=1).reshape(_H * _DH, _D).T
    Wkv3 = Wkv.reshape(_HKV, _DH, _D)
    WkvrT = jnp.concatenate([-Wkv3[:, h2:], Wkv3[:, :h2]],
                            axis=1).reshape(_HKV * _DH, _D).T
    WqT, WkvT, WoT = Wq.T, Wkv.T, Wo.T

    cq = jnp.tile(cos, (1, _H))
    sq = jnp.tile(sin, (1, _H))
    ck = jnp.tile(cos, (1, _HKV))
    sk = jnp.tile(sin, (1, _HKV))
    qw_rot = jnp.concatenate([q_norm_w[h2:], q_norm_w[:h2]])
    kw_rot = jnp.concatenate([k_norm_w[h2:], k_norm_w[:h2]])
    wq = jnp.tile(q_norm_w, _H)[None, :]
    wqr = jnp.tile(qw_rot, _H)[None, :]
    wk = jnp.tile(k_norm_w, _HKV)[None, :]
    wkr = jnp.tile(kw_rot, _HKV)[None, :]

    eq = jnp.asarray(np.kron(np.eye(_H), np.ones((_DH, 1))) / _DH, f32)
    exq = jnp.asarray(np.kron(np.eye(_H), np.ones((1, _DH))), f32)
    ek = jnp.asarray(np.kron(np.eye(_HKV), np.ones((_DH, 1))) / _DH, f32)
    exk = jnp.asarray(np.kron(np.eye(_HKV), np.ones((1, _DH))), f32)

    full = lambda shape: pl.BlockSpec(shape, lambda *_: tuple(0 for _ in shape))
    rowblk = lambda w: pl.BlockSpec((_BT, w), lambda i: (i, 0))

    q, k, v = pl.pallas_call(
        _proj_body,
        grid=(_T // _BT,),
        in_specs=[
            rowblk(_D),
            full((_D, _H * _DH)),
            full((_D, _H * _DH)),
            full((_D, _HKV * _DH)),
            full((_D, _HKV * _DH)),
            rowblk(_H * _DH),
            rowblk(_H * _DH),
            rowblk(_HKV * _DH),
            rowblk(_HKV * _DH),
            full((1, _H * _DH)),
            full((1, _H * _DH)),
            full((1, _HKV * _DH)),
            full((1, _HKV * _DH)),
            full((_H * _DH, _H)),
            full((_H, _H * _DH)),
            full((_HKV * _DH, _HKV)),
            full((_HKV, _HKV * _DH)),
        ],
        out_specs=[
            rowblk(_H * _DH),
            rowblk(_HKV * _DH),
            rowblk(_HKV * _DH),
        ],
        out_shape=[
            jax.ShapeDtypeStruct((_T, _H * _DH), jnp.bfloat16),
            jax.ShapeDtypeStruct((_T, _HKV * _DH), jnp.bfloat16),
            jax.ShapeDtypeStruct((_T, _HKV * _DH), jnp.bfloat16),
        ],
    )(hs, WqT, WqrT, WkvT, WkvrT, cq, sq, ck, sk,
      wq, wqr, wk, wkr, eq, exq, ek, exk)

    attn = pl.pallas_call(
        _attn_body,
        grid=(_H // _HPP, _T // _BQ),
        in_specs=[
            pl.BlockSpec((_BQ, _HPP * _DH), lambda j, i: (i, j)),
            pl.BlockSpec((_T, 2 * _DH), lambda j, i: (0, j)),
            pl.BlockSpec((_T, 2 * _DH), lambda j, i: (0, j)),
        ],
        out_specs=pl.BlockSpec((_BQ, _HPP * _DH), lambda j, i: (i, j)),
        out_shape=jax.ShapeDtypeStruct((_T, _H * _DH), jnp.bfloat16),
    )(q, k, v)

    out = pl.pallas_call(
        _oproj_body,
        grid=(_T // _BT,),
        in_specs=[rowblk(_H * _DH), full((_H * _DH, _D))],
        out_specs=rowblk(_D),
        out_shape=jax.ShapeDtypeStruct((_T, _D), f32),
    )(attn, WoT)

    return out.reshape(_B, _T, _D)


# speedup vs baseline: 66.2492x; 1.1923x over previous
import numpy as np
import jax
import jax.numpy as jnp
from jax.experimental import pallas as pl

_B, _T, _D = 1, 2048, 1024
_H, _HKV, _DH = 16, 8, 64
_SINK, _WIN, _TOPK = 16, 256, 256
_EPS = 1e-06
_SCALE = _DH ** -0.5
_NEG = float(np.finfo(np.float32).min)

_BT = 256
_BQ = 256
_NIT = 17
_HPP = 4


def _proj_body(hs_ref, wqT_ref, wqrT_ref, wkvT_ref, wkvrT_ref,
               cq_ref, sq_ref, ck_ref, sk_ref,
               wq_ref, wqr_ref, wk_ref, wkr_ref,
               eq_ref, exq_ref, ek_ref, exk_ref,
               q_ref, k_ref, v_ref):
    hs = hs_ref[...].astype(jnp.bfloat16)
    hp = jax.lax.Precision.HIGHEST
    qa = jnp.dot(hs, wqT_ref[...].astype(jnp.bfloat16),
                 preferred_element_type=jnp.float32)
    qb = jnp.dot(hs, wqrT_ref[...].astype(jnp.bfloat16),
                 preferred_element_type=jnp.float32)
    var_q = jnp.dot(qa * qa, eq_ref[...], preferred_element_type=jnp.float32,
                    precision=hp)
    rs_q = jnp.dot(jax.lax.rsqrt(var_q + _EPS), exq_ref[...],
                   preferred_element_type=jnp.float32, precision=hp)
    q_ref[...] = (rs_q * (qa * wq_ref[...] * cq_ref[...] +
                          qb * wqr_ref[...] * sq_ref[...])).astype(jnp.bfloat16)
    ka = jnp.dot(hs, wkvT_ref[...].astype(jnp.bfloat16),
                 preferred_element_type=jnp.float32)
    kb = jnp.dot(hs, wkvrT_ref[...].astype(jnp.bfloat16),
                 preferred_element_type=jnp.float32)
    var_k = jnp.dot(ka * ka, ek_ref[...], preferred_element_type=jnp.float32,
                    precision=hp)
    rs_k = jnp.dot(jax.lax.rsqrt(var_k + _EPS), exk_ref[...],
                   preferred_element_type=jnp.float32, precision=hp)
    k_ref[...] = (rs_k * (ka * wk_ref[...] * ck_ref[...] +
                          kb * wkr_ref[...] * sk_ref[...])).astype(jnp.bfloat16)
    v_ref[...] = ka.astype(jnp.bfloat16)


def _attn_body(q_ref, k_ref, v_ref, o_ref):
    row0 = pl.program_id(1) * _BQ
    rows = row0 + jax.lax.broadcasted_iota(jnp.int32, (_BQ, _T), 0)
    cols = jax.lax.broadcasted_iota(jnp.int32, (_BQ, _T), 1)
    base_keep = (cols < _SINK) | (jnp.abs(rows - cols) <= _WIN)
    causal = cols <= rows
    k2 = k_ref[...]
    v2 = v_ref[...]
    ss, ms = [], []
    for a in range(_HPP):
        qh = q_ref[:, a * _DH:(a + 1) * _DH]
        kh = k2[:, (a // 2) * _DH:(a // 2) * _DH + _DH]
        s = jax.lax.dot_general(qh, kh, (((1,), (1,)), ((), ())),
                                preferred_element_type=jnp.float32) * _SCALE
        ss.append(s)
        ms.append(jnp.max(s, axis=1, keepdims=True))

    def search():
        def bs(_, c):
            out = []
            for a in range(_HPP):
                lo_, hi_ = c[2 * a], c[2 * a + 1]
                mid = 0.5 * (lo_ + hi_)
                cnt = jnp.sum((ss[a] > mid).astype(jnp.float32), axis=1,
                              keepdims=True)
                pred = cnt >= _TOPK
                out.append(jnp.where(pred, mid, lo_))
                out.append(jnp.where(pred, hi_, mid))
            return tuple(out)

        init = []
        for a in range(_HPP):
            init.append(ms[a] - 18.0)
            init.append(ms[a])
        fin = jax.lax.fori_loop(0, _NIT, bs, tuple(init))
        return tuple(fin[2 * a] for a in range(_HPP))

    los = jax.lax.cond(
        pl.program_id(1) == 0,
        lambda: tuple(jnp.full((_BQ, 1), jnp.inf, jnp.float32)
                      for _ in range(_HPP)),
        search)

    for a in range(_HPP):
        s = ss[a]
        keep = (base_keep | (s > los[a])) & causal
        p = jnp.where(keep, jnp.exp(s - ms[a]), 0.0)
        p = p / jnp.sum(p, axis=1, keepdims=True)
        vh = v2[:, (a // 2) * _DH:(a // 2) * _DH + _DH]
        o_ref[:, a * _DH:(a + 1) * _DH] = jax.lax.dot_general(
            p.astype(jnp.bfloat16), vh, (((1,), (0,)), ((), ())),
            preferred_element_type=jnp.float32).astype(jnp.bfloat16)


def _oproj_body(x_ref, woT_ref, o_ref):
    o_ref[...] = jnp.dot(x_ref[...], woT_ref[...].astype(jnp.bfloat16),
                         preferred_element_type=jnp.float32)


def kernel(hidden_states, cos, sin, Wq, Wkv, Wo, q_norm_w, k_norm_w):
    f32 = jnp.float32
    hs = hidden_states.reshape(_T, _D)

    h2 = _DH // 2
    Wq3 = Wq.reshape(_H, _DH, _D)
    WqrT = jnp.concatenate([-Wq3[:, h2:], Wq3[:, :h2]],
                           axis=1).reshape(_H * _DH, _D).T
    Wkv3 = Wkv.reshape(_HKV, _DH, _D)
    WkvrT = jnp.concatenate([-Wkv3[:, h2:], Wkv3[:, :h2]],
                            axis=1).reshape(_HKV * _DH, _D).T
    WqT, WkvT, WoT = Wq.T, Wkv.T, Wo.T

    cq = jnp.tile(cos, (1, _H))
    sq = jnp.tile(sin, (1, _H))
    ck = jnp.tile(cos, (1, _HKV))
    sk = jnp.tile(sin, (1, _HKV))
    qw_rot = jnp.concatenate([q_norm_w[h2:], q_norm_w[:h2]])
    kw_rot = jnp.concatenate([k_norm_w[h2:], k_norm_w[:h2]])
    wq = jnp.tile(q_norm_w, _H)[None, :]
    wqr = jnp.tile(qw_rot, _H)[None, :]
    wk = jnp.tile(k_norm_w, _HKV)[None, :]
    wkr = jnp.tile(kw_rot, _HKV)[None, :]

    eq = jnp.asarray(np.kron(np.eye(_H), np.ones((_DH, 1))) / _DH, f32)
    exq = jnp.asarray(np.kron(np.eye(_H), np.ones((1, _DH))), f32)
    ek = jnp.asarray(np.kron(np.eye(_HKV), np.ones((_DH, 1))) / _DH, f32)
    exk = jnp.asarray(np.kron(np.eye(_HKV), np.ones((1, _DH))), f32)

    full = lambda shape: pl.BlockSpec(shape, lambda *_: tuple(0 for _ in shape))
    rowblk = lambda w: pl.BlockSpec((_BT, w), lambda i: (i, 0))

    q, k, v = pl.pallas_call(
        _proj_body,
        grid=(_T // _BT,),
        in_specs=[
            rowblk(_D),
            full((_D, _H * _DH)),
            full((_D, _H * _DH)),
            full((_D, _HKV * _DH)),
            full((_D, _HKV * _DH)),
            rowblk(_H * _DH),
            rowblk(_H * _DH),
            rowblk(_HKV * _DH),
            rowblk(_HKV * _DH),
            full((1, _H * _DH)),
            full((1, _H * _DH)),
            full((1, _HKV * _DH)),
            full((1, _HKV * _DH)),
            full((_H * _DH, _H)),
            full((_H, _H * _DH)),
            full((_HKV * _DH, _HKV)),
            full((_HKV, _HKV * _DH)),
        ],
        out_specs=[
            rowblk(_H * _DH),
            rowblk(_HKV * _DH),
            rowblk(_HKV * _DH),
        ],
        out_shape=[
            jax.ShapeDtypeStruct((_T, _H * _DH), jnp.bfloat16),
            jax.ShapeDtypeStruct((_T, _HKV * _DH), jnp.bfloat16),
            jax.ShapeDtypeStruct((_T, _HKV * _DH), jnp.bfloat16),
        ],
    )(hs, WqT, WqrT, WkvT, WkvrT, cq, sq, ck, sk,
      wq, wqr, wk, wkr, eq, exq, ek, exk)

    attn = pl.pallas_call(
        _attn_body,
        grid=(_H // _HPP, _T // _BQ),
        in_specs=[
            pl.BlockSpec((_BQ, _HPP * _DH), lambda j, i: (i, j)),
            pl.BlockSpec((_T, 2 * _DH), lambda j, i: (0, j)),
            pl.BlockSpec((_T, 2 * _DH), lambda j, i: (0, j)),
        ],
        out_specs=pl.BlockSpec((_BQ, _HPP * _DH), lambda j, i: (i, j)),
        out_shape=jax.ShapeDtypeStruct((_T, _H * _DH), jnp.bfloat16),
    )(q, k, v)

    out = pl.pallas_call(
        _oproj_body,
        grid=(_T // _BT,),
        in_specs=[rowblk(_H * _DH), full((_H * _DH, _D))],
        out_specs=rowblk(_D),
        out_shape=jax.ShapeDtypeStruct((_T, _D), f32),
    )(attn, WoT)

    return out.reshape(_B, _T, _D)


# speedup vs baseline: 68.6818x; 1.0367x over previous
import numpy as np
import jax
import jax.numpy as jnp
from jax.experimental import pallas as pl

_B, _T, _D = 1, 2048, 1024
_H, _HKV, _DH = 16, 8, 64
_SINK, _WIN, _TOPK = 16, 256, 256
_EPS = 1e-06
_SCALE = _DH ** -0.5
_NEG = float(np.finfo(np.float32).min)

_BT = 256
_BQ = 256
_NIT = 16
_HPP = 4


def _proj_body(hs_ref, wqT_ref, wqrT_ref, wkvT_ref, wkvrT_ref,
               cq_ref, sq_ref, ck_ref, sk_ref,
               wq_ref, wqr_ref, wk_ref, wkr_ref,
               eq_ref, exq_ref, ek_ref, exk_ref,
               q_ref, k_ref, v_ref):
    hs = hs_ref[...].astype(jnp.bfloat16)
    hp = jax.lax.Precision.HIGHEST
    qa = jnp.dot(hs, wqT_ref[...].astype(jnp.bfloat16),
                 preferred_element_type=jnp.float32)
    qb = jnp.dot(hs, wqrT_ref[...].astype(jnp.bfloat16),
                 preferred_element_type=jnp.float32)
    var_q = jnp.dot(qa * qa, eq_ref[...], preferred_element_type=jnp.float32,
                    precision=hp)
    rs_q = jnp.dot(jax.lax.rsqrt(var_q + _EPS), exq_ref[...],
                   preferred_element_type=jnp.float32, precision=hp)
    q_ref[...] = (rs_q * _SCALE * (qa * wq_ref[...] * cq_ref[...] +
                                   qb * wqr_ref[...] * sq_ref[...])
                  ).astype(jnp.bfloat16)
    ka = jnp.dot(hs, wkvT_ref[...].astype(jnp.bfloat16),
                 preferred_element_type=jnp.float32)
    kb = jnp.dot(hs, wkvrT_ref[...].astype(jnp.bfloat16),
                 preferred_element_type=jnp.float32)
    var_k = jnp.dot(ka * ka, ek_ref[...], preferred_element_type=jnp.float32,
                    precision=hp)
    rs_k = jnp.dot(jax.lax.rsqrt(var_k + _EPS), exk_ref[...],
                   preferred_element_type=jnp.float32, precision=hp)
    k_ref[...] = (rs_k * (ka * wk_ref[...] * ck_ref[...] +
                          kb * wkr_ref[...] * sk_ref[...])).astype(jnp.bfloat16)
    v_ref[...] = ka.astype(jnp.bfloat16)


def _attn_body(q_ref, k_ref, v_ref, o_ref):
    row0 = pl.program_id(1) * _BQ
    rows = row0 + jax.lax.broadcasted_iota(jnp.int32, (_BQ, _T), 0)
    cols = jax.lax.broadcasted_iota(jnp.int32, (_BQ, _T), 1)
    base_keep = (cols < _SINK) | (jnp.abs(rows - cols) <= _WIN)
    causal = cols <= rows
    k2 = k_ref[...]
    v2 = v_ref[...]
    ss, ms = [], []
    for a in range(_HPP):
        qh = q_ref[:, a * _DH:(a + 1) * _DH]
        kh = k2[:, (a // 2) * _DH:(a // 2) * _DH + _DH]
        s = jax.lax.dot_general(qh, kh, (((1,), (1,)), ((), ())),
                                preferred_element_type=jnp.float32)
        ss.append(s)
        ms.append(jnp.max(s, axis=1, keepdims=True))

    def search():
        def bs(_, c):
            out = []
            for a in range(_HPP):
                lo_, hi_ = c[2 * a], c[2 * a + 1]
                mid = 0.5 * (lo_ + hi_)
                cnt = jnp.sum((ss[a] > mid).astype(jnp.float32), axis=1,
                              keepdims=True)
                pred = cnt >= _TOPK
                out.append(jnp.where(pred, mid, lo_))
                out.append(jnp.where(pred, hi_, mid))
            return tuple(out)

        init = []
        for a in range(_HPP):
            init.append(ms[a] - 18.0)
            init.append(ms[a])
        fin = jax.lax.fori_loop(0, _NIT, bs, tuple(init))
        return tuple(fin[2 * a] for a in range(_HPP))

    los = jax.lax.cond(
        pl.program_id(1) == 0,
        lambda: tuple(jnp.full((_BQ, 1), jnp.inf, jnp.float32)
                      for _ in range(_HPP)),
        search)

    for a in range(_HPP):
        s = ss[a]
        keep = (base_keep | (s > los[a])) & causal
        p = jnp.where(keep, jnp.exp(s - ms[a]), 0.0)
        p = p / jnp.sum(p, axis=1, keepdims=True)
        vh = v2[:, (a // 2) * _DH:(a // 2) * _DH + _DH]
        o_ref[:, a * _DH:(a + 1) * _DH] = jax.lax.dot_general(
            p.astype(jnp.bfloat16), vh, (((1,), (0,)), ((), ())),
            preferred_element_type=jnp.float32).astype(jnp.bfloat16)


def _oproj_body(x_ref, woT_ref, o_ref):
    o_ref[...] = jnp.dot(x_ref[...], woT_ref[...].astype(jnp.bfloat16),
                         preferred_element_type=jnp.float32)


def kernel(hidden_states, cos, sin, Wq, Wkv, Wo, q_norm_w, k_norm_w):
    f32 = jnp.float32
    hs = hidden_states.reshape(_T, _D)

    h2 = _DH // 2
    Wq3 = Wq.reshape(_H, _DH, _D)
    WqrT = jnp.concatenate([-Wq3[:, h2:], Wq3[:, :h2]],
                           axis=1).reshape(_H * _DH, _D).T
    Wkv3 = Wkv.reshape(_HKV, _DH, _D)
    WkvrT = jnp.concatenate([-Wkv3[:, h2:], Wkv3[:, :h2]],
                            axis=1).reshape(_HKV * _DH, _D).T
    WqT, WkvT, WoT = Wq.T, Wkv.T, Wo.T

    cq = jnp.tile(cos, (1, _H))
    sq = jnp.tile(sin, (1, _H))
    ck = jnp.tile(cos, (1, _HKV))
    sk = jnp.tile(sin, (1, _HKV))
    qw_rot = jnp.concatenate([q_norm_w[h2:], q_norm_w[:h2]])
    kw_rot = jnp.concatenate([k_norm_w[h2:], k_norm_w[:h2]])
    wq = jnp.tile(q_norm_w, _H)[None, :]
    wqr = jnp.tile(qw_rot, _H)[None, :]
    wk = jnp.tile(k_norm_w, _HKV)[None, :]
    wkr = jnp.tile(kw_rot, _HKV)[None, :]

    eq = jnp.asarray(np.kron(np.eye(_H), np.ones((_DH, 1))) / _DH, f32)
    exq = jnp.asarray(np.kron(np.eye(_H), np.ones((1, _DH))), f32)
    ek = jnp.asarray(np.kron(np.eye(_HKV), np.ones((_DH, 1))) / _DH, f32)
    exk = jnp.asarray(np.kron(np.eye(_HKV), np.ones((1, _DH))), f32)

    full = lambda shape: pl.BlockSpec(shape, lambda *_: tuple(0 for _ in shape))
    rowblk = lambda w: pl.BlockSpec((_BT, w), lambda i: (i, 0))

    q, k, v = pl.pallas_call(
        _proj_body,
        grid=(_T // _BT,),
        in_specs=[
            rowblk(_D),
            full((_D, _H * _DH)),
            full((_D, _H * _DH)),
            full((_D, _HKV * _DH)),
            full((_D, _HKV * _DH)),
            rowblk(_H * _DH),
            rowblk(_H * _DH),
            rowblk(_HKV * _DH),
            rowblk(_HKV * _DH),
            full((1, _H * _DH)),
            full((1, _H * _DH)),
            full((1, _HKV * _DH)),
            full((1, _HKV * _DH)),
            full((_H * _DH, _H)),
            full((_H, _H * _DH)),
            full((_HKV * _DH, _HKV)),
            full((_HKV, _HKV * _DH)),
        ],
        out_specs=[
            rowblk(_H * _DH),
            rowblk(_HKV * _DH),
            rowblk(_HKV * _DH),
        ],
        out_shape=[
            jax.ShapeDtypeStruct((_T, _H * _DH), jnp.bfloat16),
            jax.ShapeDtypeStruct((_T, _HKV * _DH), jnp.bfloat16),
            jax.ShapeDtypeStruct((_T, _HKV * _DH), jnp.bfloat16),
        ],
    )(hs, WqT, WqrT, WkvT, WkvrT, cq, sq, ck, sk,
      wq, wqr, wk, wkr, eq, exq, ek, exk)

    attn = pl.pallas_call(
        _attn_body,
        grid=(_H // _HPP, _T // _BQ),
        in_specs=[
            pl.BlockSpec((_BQ, _HPP * _DH), lambda j, i: (i, j)),
            pl.BlockSpec((_T, 2 * _DH), lambda j, i: (0, j)),
            pl.BlockSpec((_T, 2 * _DH), lambda j, i: (0, j)),
        ],
        out_specs=pl.BlockSpec((_BQ, _HPP * _DH), lambda j, i: (i, j)),
        out_shape=jax.ShapeDtypeStruct((_T, _H * _DH), jnp.bfloat16),
    )(q, k, v)

    out = pl.pallas_call(
        _oproj_body,
        grid=(_T // _BT,),
        in_specs=[rowblk(_H * _DH), full((_H * _DH, _D))],
        out_specs=rowblk(_D),
        out_shape=jax.ShapeDtypeStruct((_T, _D), f32),
    )(attn, WoT)

    return out.reshape(_B, _T, _D)
